# Initial kernel scaffold; baseline (speedup 1.0000x reference)
import jax
import jax.numpy as jnp
from jax.experimental import pallas as pl


def kernel(x, edge_index, W1, b1, W2, b2, W3, b3, Wf1, bf1, Wf2, bf2):
    raise NotImplementedError("write your pallas kernel here")


# speedup vs baseline: 11.1932x; 11.1932x over previous
import functools

import jax
import jax.numpy as jnp
from jax import lax
from jax.experimental import pallas as pl
from jax.experimental.pallas import tpu as pltpu
from jax.experimental.pallas import tpu_sc as plsc

N_NODES = 100000
N_EDGES = 1600000

NC, NS, L = 2, 16, 16
N_PAD = 100352
E_PAD = 1605632
ROWS_PER_TILE = E_PAD // (NS * 128)
K_WIN = 8
N_WIN = ROWS_PER_TILE // K_WIN
NODES_PER_TILE = N_PAD // NS
R_TC = 2048
G_TC = N_PAD // R_TC


def _sc_mesh():
  return plsc.VectorSubcoreMesh(
      core_axis_name="c", subcore_axis_name="s",
      num_cores=NC, num_subcores=NS)


def _make_deg_kernel():
  @functools.partial(
      pl.kernel, mesh=_sc_mesh(),
      out_type=jax.ShapeDtypeStruct((N_PAD,), jnp.float32),
      compiler_params=pltpu.CompilerParams(use_tc_tiling_on_sc=False),
      scratch_types=[
          pltpu.VMEM((K_WIN, 128), jnp.int32),
          pltpu.VMEM((128,), jnp.float32),
          pltpu.VMEM_SHARED((N_PAD,), jnp.float32),
      ],
  )
  def deg_kernel(dst_hbm, ones_hbm, deg_hbm, didx, onesbuf, acc):
    cid = lax.axis_index("c")
    sid = lax.axis_index("s")

    @pl.when(cid == 0)
    def _():
      for i in range(128 // L):
        onesbuf[pl.ds(L * i, L)] = jnp.ones((L,), jnp.float32)
      sl = pl.ds(sid * NODES_PER_TILE, NODES_PER_TILE)
      pltpu.sync_copy(ones_hbm.at[sl], acc.at[sl])
      plsc.subcore_barrier()

      def wbody(w, _):
        pltpu.sync_copy(dst_hbm.at[sid, pl.ds(w * K_WIN, K_WIN)], didx)
        for r in range(K_WIN):
          pltpu.sync_copy(onesbuf, acc.at[didx.at[r]], add=True)
        return ()

      lax.fori_loop(0, N_WIN, wbody, ())
      plsc.subcore_barrier()
      pltpu.sync_copy(acc.at[sl], deg_hbm.at[sl])

  return deg_kernel


def _make_prop_kernel(nc):
  @functools.partial(
      pl.kernel, mesh=_sc_mesh(),
      out_type=jax.ShapeDtypeStruct((nc, N_PAD, L), jnp.float32),
      compiler_params=pltpu.CompilerParams(use_tc_tiling_on_sc=False),
      scratch_types=[
          pltpu.VMEM((K_WIN, 128), jnp.int32),
          pltpu.VMEM((K_WIN, 128), jnp.int32),
          pltpu.VMEM((K_WIN, 128, L), jnp.float32),
          pltpu.VMEM_SHARED((N_PAD, L), jnp.float32),
          pltpu.SemaphoreType.DMA,
      ],
  )
  def prop_kernel(g_hbm, src_hbm, dst_hbm, s_hbm, sidx, didx, rows, acc, sem):
    cid = lax.axis_index("c")
    sid = lax.axis_index("s")
    sl = pl.ds(sid * NODES_PER_TILE, NODES_PER_TILE)

    for j in range(nc):
      @pl.when(cid == j % NC)
      def _(j=j):
        pltpu.sync_copy(g_hbm.at[j, sl], acc.at[sl])
        plsc.subcore_barrier()

        def wbody(w, _):
          pltpu.sync_copy(src_hbm.at[sid, pl.ds(w * K_WIN, K_WIN)], sidx)
          pltpu.sync_copy(dst_hbm.at[sid, pl.ds(w * K_WIN, K_WIN)], didx)
          descs = []
          for r in range(K_WIN):
            descs.append(
                pltpu.async_copy(g_hbm.at[j].at[sidx.at[r]], rows.at[r], sem))
          for d in descs:
            d.wait()
          for r in range(K_WIN):
            pltpu.sync_copy(rows.at[r], acc.at[didx.at[r]], add=True)
          return ()

        lax.fori_loop(0, N_WIN, wbody, ())
        plsc.subcore_barrier()
        pltpu.sync_copy(acc.at[sl], s_hbm.at[j, sl])

  return prop_kernel


def _tc_prep_body(x_ref, deg_ref, dinv_ref, g_ref):
  dv = lax.rsqrt(deg_ref[...])
  dinv_ref[...] = dv
  g_ref[0] = x_ref[...] * dv[:, None]


def _tc_prep(x16, deg):
  return pl.pallas_call(
      _tc_prep_body,
      grid=(G_TC,),
      in_specs=[
          pl.BlockSpec((R_TC, L), lambda i: (i, 0)),
          pl.BlockSpec((R_TC,), lambda i: (i,)),
      ],
      out_specs=[
          pl.BlockSpec((R_TC,), lambda i: (i,)),
          pl.BlockSpec((1, R_TC, L), lambda i: (0, i, 0)),
      ],
      out_shape=[
          jax.ShapeDtypeStruct((N_PAD,), jnp.float32),
          jax.ShapeDtypeStruct((1, N_PAD, L), jnp.float32),
      ],
  )(x16, deg)


def _tc_layer_body(ncin, ncout, s_ref, dinv_ref, w_ref, b_ref, out_ref):
  dv = dinv_ref[...]
  acc = jnp.broadcast_to(b_ref[...][None, :], (R_TC, ncout * L)).astype(
      jnp.float32)
  for c in range(ncin):
    acc = acc + jnp.dot(s_ref[c] * dv[:, None], w_ref[c * L:(c + 1) * L, :],
                        preferred_element_type=jnp.float32)
  g = jnp.maximum(acc, 0.0) * dv[:, None]
  for c in range(ncout):
    out_ref[c] = g[:, c * L:(c + 1) * L]


def _tc_layer(s, dinv, w, b):
  ncin = s.shape[0]
  ncout = w.shape[1] // L
  return pl.pallas_call(
      functools.partial(_tc_layer_body, ncin, ncout),
      grid=(G_TC,),
      in_specs=[
          pl.BlockSpec((ncin, R_TC, L), lambda i: (0, i, 0)),
          pl.BlockSpec((R_TC,), lambda i: (i,)),
          pl.BlockSpec(w.shape, lambda i: (0, 0)),
          pl.BlockSpec(b.shape, lambda i: (0,)),
      ],
      out_specs=pl.BlockSpec((ncout, R_TC, L), lambda i: (0, i, 0)),
      out_shape=jax.ShapeDtypeStruct((ncout, N_PAD, L), jnp.float32),
  )(s, dinv, w, b)


def _tc_final_body(s_ref, dinv_ref, w3_ref, b3_ref, wf1_ref, bf1_ref,
                   wf2_ref, bf2_ref, out_ref):
  dv = dinv_ref[...]
  acc = jnp.broadcast_to(b3_ref[...][None, :], (R_TC, 128)).astype(jnp.float32)
  for c in range(8):
    acc = acc + jnp.dot(s_ref[c] * dv[:, None], w3_ref[c * L:(c + 1) * L, :],
                        preferred_element_type=jnp.float32)
  h3 = jnp.maximum(acc, 0.0)
  h4 = jnp.maximum(
      jnp.dot(h3, wf1_ref[...], preferred_element_type=jnp.float32)
      + bf1_ref[...][None, :], 0.0)
  out_ref[...] = (jnp.dot(h4, wf2_ref[...], preferred_element_type=jnp.float32)
                  + bf2_ref[...][None, :])


def _tc_final(s3, dinv, w3, b3, wf1, bf1, wf2, bf2):
  return pl.pallas_call(
      _tc_final_body,
      grid=(G_TC,),
      in_specs=[
          pl.BlockSpec((8, R_TC, L), lambda i: (0, i, 0)),
          pl.BlockSpec((R_TC,), lambda i: (i,)),
          pl.BlockSpec((128, 128), lambda i: (0, 0)),
          pl.BlockSpec((128,), lambda i: (0,)),
          pl.BlockSpec((128, 64), lambda i: (0, 0)),
          pl.BlockSpec((64,), lambda i: (0,)),
          pl.BlockSpec((64, 2), lambda i: (0, 0)),
          pl.BlockSpec((2,), lambda i: (0,)),
      ],
      out_specs=pl.BlockSpec((R_TC, 2), lambda i: (i, 0)),
      out_shape=jax.ShapeDtypeStruct((N_PAD, 2), jnp.float32),
  )(s3, dinv, w3, b3, wf1, bf1, wf2, bf2)


_DEG_KERNEL = None
_PROP_KERNELS = {}


def _deg_kernel():
  global _DEG_KERNEL
  if _DEG_KERNEL is None:
    _DEG_KERNEL = _make_deg_kernel()
  return _DEG_KERNEL


def _prop_kernel(nc):
  if nc not in _PROP_KERNELS:
    _PROP_KERNELS[nc] = _make_prop_kernel(nc)
  return _PROP_KERNELS[nc]


@jax.jit
def kernel(x, edge_index, W1, b1, W2, b2, W3, b3, Wf1, bf1, Wf2, bf2):
  src = edge_index[0].astype(jnp.int32)
  dst = edge_index[1].astype(jnp.int32)
  n_extra = E_PAD - N_EDGES
  pad_src = jnp.zeros((n_extra,), jnp.int32)
  pad_dst = N_NODES + (jnp.arange(n_extra, dtype=jnp.int32) % 64)
  src3 = jnp.concatenate([src, pad_src]).reshape(NS, ROWS_PER_TILE, 128)
  dst3 = jnp.concatenate([dst, pad_dst]).reshape(NS, ROWS_PER_TILE, 128)

  x16 = jnp.zeros((N_PAD, L), jnp.float32).at[:N_NODES, :3].set(x)
  ones = jnp.ones((N_PAD,), jnp.float32)
  w1p = jnp.zeros((L, 64), jnp.float32).at[:3, :].set(W1)

  deg = _deg_kernel()(dst3, ones)
  dinv, g1 = _tc_prep(x16, deg)
  s1 = _prop_kernel(1)(g1, src3, dst3)
  g2 = _tc_layer(s1, dinv, w1p, b1)
  s2 = _prop_kernel(4)(g2, src3, dst3)
  g3 = _tc_layer(s2, dinv, W2, b2)
  s3 = _prop_kernel(8)(g3, src3, dst3)
  out = _tc_final(s3, dinv, W3, b3, Wf1, bf1, Wf2, bf2)
  return out[:N_NODES]


# speedup vs baseline: 12.0310x; 1.0748x over previous
import functools

import jax
import jax.numpy as jnp
from jax import lax
from jax.experimental import pallas as pl
from jax.experimental.pallas import tpu as pltpu
from jax.experimental.pallas import tpu_sc as plsc

N_NODES = 100000
N_EDGES = 1600000

NC, NS, L = 2, 16, 16
N_PAD = 100352
E_PAD = 1605632
ROWS_PER_TILE = E_PAD // (NS * 128)
K_WIN = 4
N_WIN = ROWS_PER_TILE // K_WIN
NODES_PER_TILE = N_PAD // NS
R_TC = 2048
G_TC = N_PAD // R_TC


def _sc_mesh():
  return plsc.VectorSubcoreMesh(
      core_axis_name="c", subcore_axis_name="s",
      num_cores=NC, num_subcores=NS)


def _make_deg_kernel():
  @functools.partial(
      pl.kernel, mesh=_sc_mesh(),
      out_type=jax.ShapeDtypeStruct((NC, N_PAD), jnp.float32),
      compiler_params=pltpu.CompilerParams(use_tc_tiling_on_sc=False),
      scratch_types=[
          pltpu.VMEM((K_WIN, 128), jnp.int32),
          pltpu.VMEM((K_WIN, 128), jnp.int32),
          pltpu.VMEM((128,), jnp.float32),
          pltpu.VMEM_SHARED((N_PAD,), jnp.float32),
      ],
  )
  def deg_kernel(dst_hbm, ones_hbm, deg_hbm, didxa, didxb, onesbuf, acc):
    cid = lax.axis_index("c")
    sid = lax.axis_index("s")
    sl = pl.ds(sid * NODES_PER_TILE, NODES_PER_TILE)
    half = N_WIN // 2

    def scat(w, didx):
      pltpu.sync_copy(dst_hbm.at[sid, pl.ds(w * K_WIN, K_WIN)], didx)
      for r in range(K_WIN):
        pltpu.sync_copy(onesbuf, acc.at[didx.at[r]], add=True)

    def run(w_base, out_j):
      for i in range(128 // L):
        onesbuf[pl.ds(L * i, L)] = jnp.ones((L,), jnp.float32)
      pltpu.sync_copy(ones_hbm.at[sl], acc.at[sl])
      plsc.subcore_barrier()

      def pair(p, _):
        scat(w_base + 2 * p, didxa)
        scat(w_base + 2 * p + 1, didxb)
        return ()

      lax.fori_loop(0, half // 2, pair, ())
      if half % 2:
        scat(w_base + half - 1, didxa)
      plsc.subcore_barrier()
      pltpu.sync_copy(acc.at[sl], deg_hbm.at[out_j, sl])

    @pl.when(cid == 0)
    def _():
      run(0, 0)

    @pl.when(cid == 1)
    def _():
      run(half, 1)

  return deg_kernel


def _make_prop_kernel(nc):
  n_out = NC if nc == 1 else nc

  @functools.partial(
      pl.kernel, mesh=_sc_mesh(),
      out_type=jax.ShapeDtypeStruct((n_out, N_PAD, L), jnp.float32),
      compiler_params=pltpu.CompilerParams(use_tc_tiling_on_sc=False),
      scratch_types=[
          pltpu.VMEM((K_WIN, 128), jnp.int32),
          pltpu.VMEM((K_WIN, 128), jnp.int32),
          pltpu.VMEM((K_WIN, 128), jnp.int32),
          pltpu.VMEM((K_WIN, 128), jnp.int32),
          pltpu.VMEM((K_WIN, 128, L), jnp.float32),
          pltpu.VMEM((K_WIN, 128, L), jnp.float32),
          pltpu.VMEM_SHARED((N_PAD, L), jnp.float32),
          pltpu.SemaphoreType.DMA,
          pltpu.SemaphoreType.DMA,
      ],
  )
  def prop_kernel(g_hbm, src_hbm, dst_hbm, s_hbm,
                  sidxa, didxa, sidxb, didxb, rowsa, rowsb, acc, gsema, gsemb):
    cid = lax.axis_index("c")
    sid = lax.axis_index("s")
    sl = pl.ds(sid * NODES_PER_TILE, NODES_PER_TILE)

    def fire(jc, w, sidx, didx, rows, gsem):
      pltpu.sync_copy(src_hbm.at[sid, pl.ds(w * K_WIN, K_WIN)], sidx)
      pltpu.sync_copy(dst_hbm.at[sid, pl.ds(w * K_WIN, K_WIN)], didx)
      return [pltpu.async_copy(g_hbm.at[jc].at[sidx.at[r]], rows.at[r], gsem)
              for r in range(K_WIN)]

    def drain_scatter(descs, didx, rows):
      for d in descs:
        d.wait()
      for r in range(K_WIN):
        pltpu.sync_copy(rows.at[r], acc.at[didx.at[r]], add=True)

    def run(jc, w_base, n_win, out_j):
      pltpu.sync_copy(g_hbm.at[jc, sl], acc.at[sl])
      plsc.subcore_barrier()

      def pair(p, _):
        wa = w_base + 2 * p
        da = fire(jc, wa, sidxa, didxa, rowsa, gsema)
        db = fire(jc, wa + 1, sidxb, didxb, rowsb, gsemb)
        drain_scatter(da, didxa, rowsa)
        drain_scatter(db, didxb, rowsb)
        return ()

      lax.fori_loop(0, n_win // 2, pair, ())
      if n_win % 2:
        dt = fire(jc, w_base + n_win - 1, sidxa, didxa, rowsa, gsema)
        drain_scatter(dt, didxa, rowsa)
      plsc.subcore_barrier()
      pltpu.sync_copy(acc.at[sl], s_hbm.at[out_j, sl])

    if nc == 1:
      half = N_WIN // 2

      @pl.when(cid == 0)
      def _():
        run(0, 0, half, 0)

      @pl.when(cid == 1)
      def _():
        run(0, half, N_WIN - half, 1)
    else:
      for j in range(nc):
        @pl.when(cid == j % NC)
        def _(j=j):
          run(j, 0, N_WIN, j)

  return prop_kernel


def _tc_prep_body(x_ref, deg_ref, dinv_ref, g_ref):
  dv = lax.rsqrt(deg_ref[0] + deg_ref[1] - 1.0)
  dinv_ref[...] = dv
  g_ref[0] = x_ref[...] * dv[:, None]


def _tc_prep(x16, deg):
  return pl.pallas_call(
      _tc_prep_body,
      grid=(G_TC,),
      in_specs=[
          pl.BlockSpec((R_TC, L), lambda i: (i, 0)),
          pl.BlockSpec((NC, R_TC), lambda i: (0, i)),
      ],
      out_specs=[
          pl.BlockSpec((R_TC,), lambda i: (i,)),
          pl.BlockSpec((1, R_TC, L), lambda i: (0, i, 0)),
      ],
      out_shape=[
          jax.ShapeDtypeStruct((N_PAD,), jnp.float32),
          jax.ShapeDtypeStruct((1, N_PAD, L), jnp.float32),
      ],
  )(x16, deg)


def _tc_layer_body(ncin, ncout, partial, s_ref, dinv_ref, w_ref, b_ref,
                   g_ref, out_ref):
  dv = dinv_ref[...]
  acc = jnp.broadcast_to(b_ref[...][None, :], (R_TC, ncout * L)).astype(
      jnp.float32)
  for c in range(ncin):
    if partial:
      sc = s_ref[0] + s_ref[1] - g_ref[0]
    else:
      sc = s_ref[c]
    acc = acc + jnp.dot(sc * dv[:, None], w_ref[c * L:(c + 1) * L, :],
                        preferred_element_type=jnp.float32)
  g = jnp.maximum(acc, 0.0) * dv[:, None]
  for c in range(ncout):
    out_ref[c] = g[:, c * L:(c + 1) * L]


def _tc_layer(s, dinv, w, b, g1=None):
  partial = g1 is not None
  ncin = 1 if partial else s.shape[0]
  ncout = w.shape[1] // L
  nphys = s.shape[0]
  in_specs = [
      pl.BlockSpec((nphys, R_TC, L), lambda i: (0, i, 0)),
      pl.BlockSpec((R_TC,), lambda i: (i,)),
      pl.BlockSpec(w.shape, lambda i: (0, 0)),
      pl.BlockSpec(b.shape, lambda i: (0,)),
  ]
  args = [s, dinv, w, b]
  if partial:
    in_specs.append(pl.BlockSpec((1, R_TC, L), lambda i: (0, i, 0)))
    args.append(g1)
  else:
    in_specs.append(pl.BlockSpec((1, 8), lambda i: (0, 0)))
    args.append(jnp.zeros((1, 8), jnp.float32))
  return pl.pallas_call(
      functools.partial(_tc_layer_body, ncin, ncout, partial),
      grid=(G_TC,),
      in_specs=in_specs,
      out_specs=pl.BlockSpec((ncout, R_TC, L), lambda i: (0, i, 0)),
      out_shape=jax.ShapeDtypeStruct((ncout, N_PAD, L), jnp.float32),
  )(*args)


def _tc_final_body(s_ref, dinv_ref, w3_ref, b3_ref, wf1_ref, bf1_ref,
                   wf2_ref, bf2_ref, out_ref):
  dv = dinv_ref[...]
  acc = jnp.broadcast_to(b3_ref[...][None, :], (R_TC, 128)).astype(jnp.float32)
  for c in range(8):
    acc = acc + jnp.dot(s_ref[c] * dv[:, None], w3_ref[c * L:(c + 1) * L, :],
                        preferred_element_type=jnp.float32)
  h3 = jnp.maximum(acc, 0.0)
  h4 = jnp.maximum(
      jnp.dot(h3, wf1_ref[...], preferred_element_type=jnp.float32)
      + bf1_ref[...][None, :], 0.0)
  out_ref[...] = (jnp.dot(h4, wf2_ref[...], preferred_element_type=jnp.float32)
                  + bf2_ref[...][None, :])


def _tc_final(s3, dinv, w3, b3, wf1, bf1, wf2, bf2):
  return pl.pallas_call(
      _tc_final_body,
      grid=(G_TC,),
      in_specs=[
          pl.BlockSpec((8, R_TC, L), lambda i: (0, i, 0)),
          pl.BlockSpec((R_TC,), lambda i: (i,)),
          pl.BlockSpec((128, 128), lambda i: (0, 0)),
          pl.BlockSpec((128,), lambda i: (0,)),
          pl.BlockSpec((128, 64), lambda i: (0, 0)),
          pl.BlockSpec((64,), lambda i: (0,)),
          pl.BlockSpec((64, 2), lambda i: (0, 0)),
          pl.BlockSpec((2,), lambda i: (0,)),
      ],
      out_specs=pl.BlockSpec((R_TC, 2), lambda i: (i, 0)),
      out_shape=jax.ShapeDtypeStruct((N_PAD, 2), jnp.float32),
  )(s3, dinv, w3, b3, wf1, bf1, wf2, bf2)


_DEG_KERNEL = None
_PROP_KERNELS = {}


def _deg_kernel():
  global _DEG_KERNEL
  if _DEG_KERNEL is None:
    _DEG_KERNEL = _make_deg_kernel()
  return _DEG_KERNEL


def _prop_kernel(nc):
  if nc not in _PROP_KERNELS:
    _PROP_KERNELS[nc] = _make_prop_kernel(nc)
  return _PROP_KERNELS[nc]


@jax.jit
def kernel(x, edge_index, W1, b1, W2, b2, W3, b3, Wf1, bf1, Wf2, bf2):
  src = edge_index[0].astype(jnp.int32)
  dst = edge_index[1].astype(jnp.int32)
  n_extra = E_PAD - N_EDGES
  pad_src = jnp.zeros((n_extra,), jnp.int32)
  pad_dst = N_NODES + (jnp.arange(n_extra, dtype=jnp.int32) % 64)
  src3 = jnp.concatenate([src, pad_src]).reshape(NS, ROWS_PER_TILE, 128)
  dst3 = jnp.concatenate([dst, pad_dst]).reshape(NS, ROWS_PER_TILE, 128)

  x16 = jnp.zeros((N_PAD, L), jnp.float32).at[:N_NODES, :3].set(x)
  ones = jnp.ones((N_PAD,), jnp.float32)
  w1p = jnp.zeros((L, 64), jnp.float32).at[:3, :].set(W1)

  deg = _deg_kernel()(dst3, ones)
  dinv, g1 = _tc_prep(x16, deg)
  s1 = _prop_kernel(1)(g1, src3, dst3)
  g2 = _tc_layer(s1, dinv, w1p, b1, g1=g1)
  s2 = _prop_kernel(4)(g2, src3, dst3)
  g3 = _tc_layer(s2, dinv, W2, b2)
  s3 = _prop_kernel(8)(g3, src3, dst3)
  out = _tc_final(s3, dinv, W3, b3, Wf1, bf1, Wf2, bf2)
  return out[:N_NODES]


# speedup vs baseline: 12.9853x; 1.0793x over previous
import functools

import jax
import jax.numpy as jnp
from jax import lax
from jax.experimental import pallas as pl
from jax.experimental.pallas import tpu as pltpu
from jax.experimental.pallas import tpu_sc as plsc

N_NODES = 100000
N_EDGES = 1600000

NC, NS, L = 2, 16, 16
N_PAD = 100352
E_PAD = 1605632
ROWS_PER_TILE = E_PAD // (NS * 128)
K_WIN = 4
N_WIN = ROWS_PER_TILE // K_WIN
NODES_PER_TILE = N_PAD // NS
R_TC = 2048
G_TC = N_PAD // R_TC


def _sc_mesh():
  return plsc.VectorSubcoreMesh(
      core_axis_name="c", subcore_axis_name="s",
      num_cores=NC, num_subcores=NS)


def _make_deg_kernel():
  @functools.partial(
      pl.kernel, mesh=_sc_mesh(),
      out_type=jax.ShapeDtypeStruct((NC, N_PAD), jnp.float32),
      compiler_params=pltpu.CompilerParams(use_tc_tiling_on_sc=False),
      scratch_types=[
          pltpu.VMEM((K_WIN, 128), jnp.int32),
          pltpu.VMEM((K_WIN, 128), jnp.int32),
          pltpu.VMEM((K_WIN, 128), jnp.float32),
          pltpu.VMEM_SHARED((N_PAD,), jnp.float32),
      ],
  )
  def deg_kernel(dst_hbm, ones_hbm, deg_hbm, didxa, didxb, onesbuf, acc):
    cid = lax.axis_index("c")
    sid = lax.axis_index("s")
    sl = pl.ds(sid * NODES_PER_TILE, NODES_PER_TILE)
    half = N_WIN // 2

    def scat(w, didx):
      pltpu.sync_copy(dst_hbm.at[sid, pl.ds(w * K_WIN, K_WIN)], didx)
      for r in range(K_WIN):
        pltpu.sync_copy(onesbuf.at[r], acc.at[didx.at[r]], add=True)

    def run(w_base, out_j):
      for k in range(K_WIN):
        for i in range(128 // L):
          onesbuf[k, pl.ds(L * i, L)] = jnp.ones((L,), jnp.float32)
      pltpu.sync_copy(ones_hbm.at[sl], acc.at[sl])
      plsc.subcore_barrier()

      def pair(p, _):
        scat(w_base + 2 * p, didxa)
        scat(w_base + 2 * p + 1, didxb)
        return ()

      lax.fori_loop(0, half // 2, pair, ())
      if half % 2:
        scat(w_base + half - 1, didxa)
      plsc.subcore_barrier()
      pltpu.sync_copy(acc.at[sl], deg_hbm.at[out_j, sl])

    @pl.when(cid == 0)
    def _():
      run(0, 0)

    @pl.when(cid == 1)
    def _():
      run(half, 1)

  return deg_kernel


def _make_prop_kernel(nc):
  n_out = NC if nc == 1 else nc

  @functools.partial(
      pl.kernel, mesh=_sc_mesh(),
      out_type=jax.ShapeDtypeStruct((n_out, N_PAD, L), jnp.float32),
      compiler_params=pltpu.CompilerParams(use_tc_tiling_on_sc=False),
      scratch_types=[
          pltpu.VMEM((K_WIN, 128), jnp.int32),
          pltpu.VMEM((K_WIN, 128), jnp.int32),
          pltpu.VMEM((K_WIN, 128), jnp.int32),
          pltpu.VMEM((K_WIN, 128), jnp.int32),
          pltpu.VMEM((K_WIN, 128, L), jnp.float32),
          pltpu.VMEM((K_WIN, 128, L), jnp.float32),
          pltpu.VMEM_SHARED((N_PAD, L), jnp.float32),
          pltpu.SemaphoreType.DMA,
          pltpu.SemaphoreType.DMA,
          pltpu.SemaphoreType.DMA,
          pltpu.SemaphoreType.DMA,
      ],
  )
  def prop_kernel(g_hbm, src_hbm, dst_hbm, s_hbm,
                  sidxa, didxa, sidxb, didxb, rowsa, rowsb, acc,
                  gsema, gsemb, ssema, ssemb):
    cid = lax.axis_index("c")
    sid = lax.axis_index("s")
    sl = pl.ds(sid * NODES_PER_TILE, NODES_PER_TILE)

    def fire(jc, w, sidx, didx, rows, gsem):
      pltpu.sync_copy(src_hbm.at[sid, pl.ds(w * K_WIN, K_WIN)], sidx)
      pltpu.sync_copy(dst_hbm.at[sid, pl.ds(w * K_WIN, K_WIN)], didx)
      return [pltpu.async_copy(g_hbm.at[jc].at[sidx.at[r]], rows.at[r], gsem)
              for r in range(K_WIN)]

    def fire_scatter(descs, didx, rows, ssem):
      for d in descs:
        d.wait()
      for r in range(K_WIN):
        pltpu.async_copy(rows.at[r], acc.at[didx.at[r]], ssem, add=True)

    def drain_scatter(didx, rows, ssem):
      for r in range(K_WIN):
        pltpu.make_async_copy(rows.at[r], acc.at[didx.at[r]], ssem).wait()

    def run(jc, w_base, n_win, out_j):
      pltpu.sync_copy(g_hbm.at[jc, sl], acc.at[sl])
      plsc.subcore_barrier()

      da = fire(jc, w_base, sidxa, didxa, rowsa, gsema)
      db = fire(jc, w_base + 1, sidxb, didxb, rowsb, gsemb)
      fire_scatter(da, didxa, rowsa, ssema)
      fire_scatter(db, didxb, rowsb, ssemb)

      def pair(p, _):
        wa = w_base + 2 * p
        drain_scatter(didxa, rowsa, ssema)
        da = fire(jc, wa, sidxa, didxa, rowsa, gsema)
        drain_scatter(didxb, rowsb, ssemb)
        db = fire(jc, wa + 1, sidxb, didxb, rowsb, gsemb)
        fire_scatter(da, didxa, rowsa, ssema)
        fire_scatter(db, didxb, rowsb, ssemb)
        return ()

      lax.fori_loop(1, n_win // 2, pair, ())
      drain_scatter(didxa, rowsa, ssema)
      drain_scatter(didxb, rowsb, ssemb)
      plsc.subcore_barrier()
      pltpu.sync_copy(acc.at[sl], s_hbm.at[out_j, sl])

    if nc == 1:
      half = N_WIN // 2

      @pl.when(cid == 0)
      def _():
        run(0, 0, half, 0)

      @pl.when(cid == 1)
      def _():
        run(0, half, N_WIN - half, 1)
    else:
      for j in range(nc):
        @pl.when(cid == j % NC)
        def _(j=j):
          run(j, 0, N_WIN, j)

  return prop_kernel


def _tc_prep_body(x_ref, deg_ref, dinv_ref, g_ref):
  dv = lax.rsqrt(deg_ref[0] + deg_ref[1] - 1.0)
  dinv_ref[...] = dv
  g_ref[0] = x_ref[...] * dv[:, None]


def _tc_prep(x16, deg):
  return pl.pallas_call(
      _tc_prep_body,
      grid=(G_TC,),
      in_specs=[
          pl.BlockSpec((R_TC, L), lambda i: (i, 0)),
          pl.BlockSpec((NC, R_TC), lambda i: (0, i)),
      ],
      out_specs=[
          pl.BlockSpec((R_TC,), lambda i: (i,)),
          pl.BlockSpec((1, R_TC, L), lambda i: (0, i, 0)),
      ],
      out_shape=[
          jax.ShapeDtypeStruct((N_PAD,), jnp.float32),
          jax.ShapeDtypeStruct((1, N_PAD, L), jnp.float32),
      ],
  )(x16, deg)


def _tc_layer_body(ncin, ncout, partial, s_ref, dinv_ref, w_ref, b_ref,
                   g_ref, out_ref):
  dv = dinv_ref[...]
  acc = jnp.broadcast_to(b_ref[...][None, :], (R_TC, ncout * L)).astype(
      jnp.float32)
  for c in range(ncin):
    if partial:
      sc = s_ref[0] + s_ref[1] - g_ref[0]
    else:
      sc = s_ref[c]
    acc = acc + jnp.dot(sc * dv[:, None], w_ref[c * L:(c + 1) * L, :],
                        preferred_element_type=jnp.float32)
  g = jnp.maximum(acc, 0.0) * dv[:, None]
  for c in range(ncout):
    out_ref[c] = g[:, c * L:(c + 1) * L]


def _tc_layer(s, dinv, w, b, g1=None):
  partial = g1 is not None
  ncin = 1 if partial else s.shape[0]
  ncout = w.shape[1] // L
  nphys = s.shape[0]
  in_specs = [
      pl.BlockSpec((nphys, R_TC, L), lambda i: (0, i, 0)),
      pl.BlockSpec((R_TC,), lambda i: (i,)),
      pl.BlockSpec(w.shape, lambda i: (0, 0)),
      pl.BlockSpec(b.shape, lambda i: (0,)),
  ]
  args = [s, dinv, w, b]
  if partial:
    in_specs.append(pl.BlockSpec((1, R_TC, L), lambda i: (0, i, 0)))
    args.append(g1)
  else:
    in_specs.append(pl.BlockSpec((1, 8), lambda i: (0, 0)))
    args.append(jnp.zeros((1, 8), jnp.float32))
  return pl.pallas_call(
      functools.partial(_tc_layer_body, ncin, ncout, partial),
      grid=(G_TC,),
      in_specs=in_specs,
      out_specs=pl.BlockSpec((ncout, R_TC, L), lambda i: (0, i, 0)),
      out_shape=jax.ShapeDtypeStruct((ncout, N_PAD, L), jnp.float32),
  )(*args)


def _tc_final_body(s_ref, dinv_ref, w3_ref, b3_ref, wf1_ref, bf1_ref,
                   wf2_ref, bf2_ref, out_ref):
  dv = dinv_ref[...]
  acc = jnp.broadcast_to(b3_ref[...][None, :], (R_TC, 128)).astype(jnp.float32)
  for c in range(8):
    acc = acc + jnp.dot(s_ref[c] * dv[:, None], w3_ref[c * L:(c + 1) * L, :],
                        preferred_element_type=jnp.float32)
  h3 = jnp.maximum(acc, 0.0)
  h4 = jnp.maximum(
      jnp.dot(h3, wf1_ref[...], preferred_element_type=jnp.float32)
      + bf1_ref[...][None, :], 0.0)
  out_ref[...] = (jnp.dot(h4, wf2_ref[...], preferred_element_type=jnp.float32)
                  + bf2_ref[...][None, :])


def _tc_final(s3, dinv, w3, b3, wf1, bf1, wf2, bf2):
  return pl.pallas_call(
      _tc_final_body,
      grid=(G_TC,),
      in_specs=[
          pl.BlockSpec((8, R_TC, L), lambda i: (0, i, 0)),
          pl.BlockSpec((R_TC,), lambda i: (i,)),
          pl.BlockSpec((128, 128), lambda i: (0, 0)),
          pl.BlockSpec((128,), lambda i: (0,)),
          pl.BlockSpec((128, 64), lambda i: (0, 0)),
          pl.BlockSpec((64,), lambda i: (0,)),
          pl.BlockSpec((64, 2), lambda i: (0, 0)),
          pl.BlockSpec((2,), lambda i: (0,)),
      ],
      out_specs=pl.BlockSpec((R_TC, 2), lambda i: (i, 0)),
      out_shape=jax.ShapeDtypeStruct((N_PAD, 2), jnp.float32),
  )(s3, dinv, w3, b3, wf1, bf1, wf2, bf2)


_DEG_KERNEL = None
_PROP_KERNELS = {}


def _deg_kernel():
  global _DEG_KERNEL
  if _DEG_KERNEL is None:
    _DEG_KERNEL = _make_deg_kernel()
  return _DEG_KERNEL


def _prop_kernel(nc):
  if nc not in _PROP_KERNELS:
    _PROP_KERNELS[nc] = _make_prop_kernel(nc)
  return _PROP_KERNELS[nc]


@jax.jit
def kernel(x, edge_index, W1, b1, W2, b2, W3, b3, Wf1, bf1, Wf2, bf2):
  src = edge_index[0].astype(jnp.int32)
  dst = edge_index[1].astype(jnp.int32)
  n_extra = E_PAD - N_EDGES
  pad_src = jnp.zeros((n_extra,), jnp.int32)
  pad_dst = N_NODES + (jnp.arange(n_extra, dtype=jnp.int32) % 64)
  src3 = jnp.concatenate([src, pad_src]).reshape(NS, ROWS_PER_TILE, 128)
  dst3 = jnp.concatenate([dst, pad_dst]).reshape(NS, ROWS_PER_TILE, 128)

  x16 = jnp.zeros((N_PAD, L), jnp.float32).at[:N_NODES, :3].set(x)
  ones = jnp.ones((N_PAD,), jnp.float32)
  w1p = jnp.zeros((L, 64), jnp.float32).at[:3, :].set(W1)

  deg = _deg_kernel()(dst3, ones)
  dinv, g1 = _tc_prep(x16, deg)
  s1 = _prop_kernel(1)(g1, src3, dst3)
  g2 = _tc_layer(s1, dinv, w1p, b1, g1=g1)
  s2 = _prop_kernel(4)(g2, src3, dst3)
  g3 = _tc_layer(s2, dinv, W2, b2)
  s3 = _prop_kernel(8)(g3, src3, dst3)
  out = _tc_final(s3, dinv, W3, b3, Wf1, bf1, Wf2, bf2)
  return out[:N_NODES]


# speedup vs baseline: 13.6164x; 1.0486x over previous
import functools

import jax
import jax.numpy as jnp
from jax import lax
from jax.experimental import pallas as pl
from jax.experimental.pallas import tpu as pltpu
from jax.experimental.pallas import tpu_sc as plsc

N_NODES = 100000
N_EDGES = 1600000

NC, NS, L = 2, 16, 16
N_PAD = 100352
E_PAD = 1605632
ROWS_PER_TILE = E_PAD // (NS * 128)
K_WIN = 4
N_WIN = ROWS_PER_TILE // K_WIN
NODES_PER_TILE = N_PAD // NS
R_TC = 2048
G_TC = N_PAD // R_TC


def _sc_mesh():
  return plsc.VectorSubcoreMesh(
      core_axis_name="c", subcore_axis_name="s",
      num_cores=NC, num_subcores=NS)


def _make_deg_kernel():
  @functools.partial(
      pl.kernel, mesh=_sc_mesh(),
      out_type=jax.ShapeDtypeStruct((NC, N_PAD), jnp.float32),
      compiler_params=pltpu.CompilerParams(use_tc_tiling_on_sc=False),
      scratch_types=[
          pltpu.VMEM((K_WIN, 2, 128), jnp.int32),
          pltpu.VMEM((K_WIN, 2, 128), jnp.int32),
          pltpu.VMEM((K_WIN, 128), jnp.float32),
          pltpu.VMEM_SHARED((N_PAD,), jnp.float32),
      ],
  )
  def deg_kernel(sd_hbm, ones_hbm, deg_hbm, didxa, didxb, onesbuf, acc):
    cid = lax.axis_index("c")
    sid = lax.axis_index("s")
    sl = pl.ds(sid * NODES_PER_TILE, NODES_PER_TILE)
    half = N_WIN // 2

    def scat(w, didx):
      pltpu.sync_copy(sd_hbm.at[sid, pl.ds(w * K_WIN, K_WIN)], didx)
      for r in range(K_WIN):
        pltpu.sync_copy(onesbuf.at[r], acc.at[didx.at[r, 1]], add=True)

    def run(w_base, out_j):
      for k in range(K_WIN):
        for i in range(128 // L):
          onesbuf[k, pl.ds(L * i, L)] = jnp.ones((L,), jnp.float32)
      pltpu.sync_copy(ones_hbm.at[sl], acc.at[sl])
      plsc.subcore_barrier()

      def pair(p, _):
        scat(w_base + 2 * p, didxa)
        scat(w_base + 2 * p + 1, didxb)
        return ()

      lax.fori_loop(0, half // 2, pair, ())
      if half % 2:
        scat(w_base + half - 1, didxa)
      plsc.subcore_barrier()
      pltpu.sync_copy(acc.at[sl], deg_hbm.at[out_j, sl])

    @pl.when(cid == 0)
    def _():
      run(0, 0)

    @pl.when(cid == 1)
    def _():
      run(half, 1)

  return deg_kernel


def _make_prop_kernel(nc):
  n_out = NC if nc == 1 else nc

  @functools.partial(
      pl.kernel, mesh=_sc_mesh(),
      out_type=jax.ShapeDtypeStruct((n_out, N_PAD, L), jnp.float32),
      compiler_params=pltpu.CompilerParams(use_tc_tiling_on_sc=False),
      scratch_types=[
          pltpu.VMEM((K_WIN, 2, 128), jnp.int32),
          pltpu.VMEM((K_WIN, 2, 128), jnp.int32),
          pltpu.VMEM((K_WIN, 128, L), jnp.float32),
          pltpu.VMEM((K_WIN, 128, L), jnp.float32),
          pltpu.VMEM_SHARED((N_PAD, L), jnp.float32),
          pltpu.SemaphoreType.DMA,
          pltpu.SemaphoreType.DMA,
      ],
  )
  def prop_kernel(g_hbm, sd_hbm, s_hbm,
                  idxa, idxb, rowsa, rowsb, acc, gsema, gsemb):
    cid = lax.axis_index("c")
    sid = lax.axis_index("s")
    sl = pl.ds(sid * NODES_PER_TILE, NODES_PER_TILE)

    def fire(jc, w, idx, rows, gsem):
      pltpu.sync_copy(sd_hbm.at[sid, pl.ds(w * K_WIN, K_WIN)], idx)
      return [pltpu.async_copy(g_hbm.at[jc].at[idx.at[r, 0]], rows.at[r], gsem)
              for r in range(K_WIN)]

    def drain_scatter(descs, idx, rows):
      for d in descs:
        d.wait()
      for r in range(K_WIN):
        pltpu.sync_copy(rows.at[r], acc.at[idx.at[r, 1]], add=True)

    def run(jc, w_base, n_win, out_j):
      pltpu.sync_copy(g_hbm.at[jc, sl], acc.at[sl])
      plsc.subcore_barrier()

      def pair(p, _):
        wa = w_base + 2 * p
        da = fire(jc, wa, idxa, rowsa, gsema)
        db = fire(jc, wa + 1, idxb, rowsb, gsemb)
        drain_scatter(da, idxa, rowsa)
        drain_scatter(db, idxb, rowsb)
        return ()

      lax.fori_loop(0, n_win // 2, pair, ())
      plsc.subcore_barrier()
      pltpu.sync_copy(acc.at[sl], s_hbm.at[out_j, sl])

    if nc == 1:
      half = N_WIN // 2

      @pl.when(cid == 0)
      def _():
        run(0, 0, half, 0)

      @pl.when(cid == 1)
      def _():
        run(0, half, N_WIN - half, 1)
    else:
      for j in range(nc):
        @pl.when(cid == j % NC)
        def _(j=j):
          run(j, 0, N_WIN, j)

  return prop_kernel


def _tc_prep_body(x_ref, deg_ref, dinv_ref, g_ref):
  dv = lax.rsqrt(deg_ref[0] + deg_ref[1] - 1.0)
  dinv_ref[...] = dv
  g_ref[0] = x_ref[...] * dv[:, None]


def _tc_prep(x16, deg):
  return pl.pallas_call(
      _tc_prep_body,
      grid=(G_TC,),
      in_specs=[
          pl.BlockSpec((R_TC, L), lambda i: (i, 0)),
          pl.BlockSpec((NC, R_TC), lambda i: (0, i)),
      ],
      out_specs=[
          pl.BlockSpec((R_TC,), lambda i: (i,)),
          pl.BlockSpec((1, R_TC, L), lambda i: (0, i, 0)),
      ],
      out_shape=[
          jax.ShapeDtypeStruct((N_PAD,), jnp.float32),
          jax.ShapeDtypeStruct((1, N_PAD, L), jnp.float32),
      ],
  )(x16, deg)


def _tc_layer_body(ncin, ncout, partial, s_ref, dinv_ref, w_ref, b_ref,
                   g_ref, out_ref):
  dv = dinv_ref[...]
  acc = jnp.broadcast_to(b_ref[...][None, :], (R_TC, ncout * L)).astype(
      jnp.float32)
  for c in range(ncin):
    if partial:
      sc = s_ref[0] + s_ref[1] - g_ref[0]
    else:
      sc = s_ref[c]
    acc = acc + jnp.dot(sc * dv[:, None], w_ref[c * L:(c + 1) * L, :],
                        preferred_element_type=jnp.float32)
  g = jnp.maximum(acc, 0.0) * dv[:, None]
  for c in range(ncout):
    out_ref[c] = g[:, c * L:(c + 1) * L]


def _tc_layer(s, dinv, w, b, g1=None):
  partial = g1 is not None
  ncin = 1 if partial else s.shape[0]
  ncout = w.shape[1] // L
  nphys = s.shape[0]
  in_specs = [
      pl.BlockSpec((nphys, R_TC, L), lambda i: (0, i, 0)),
      pl.BlockSpec((R_TC,), lambda i: (i,)),
      pl.BlockSpec(w.shape, lambda i: (0, 0)),
      pl.BlockSpec(b.shape, lambda i: (0,)),
  ]
  args = [s, dinv, w, b]
  if partial:
    in_specs.append(pl.BlockSpec((1, R_TC, L), lambda i: (0, i, 0)))
    args.append(g1)
  else:
    in_specs.append(pl.BlockSpec((1, 8), lambda i: (0, 0)))
    args.append(jnp.zeros((1, 8), jnp.float32))
  return pl.pallas_call(
      functools.partial(_tc_layer_body, ncin, ncout, partial),
      grid=(G_TC,),
      in_specs=in_specs,
      out_specs=pl.BlockSpec((ncout, R_TC, L), lambda i: (0, i, 0)),
      out_shape=jax.ShapeDtypeStruct((ncout, N_PAD, L), jnp.float32),
  )(*args)


def _tc_final_body(s_ref, dinv_ref, w3_ref, b3_ref, wf1_ref, bf1_ref,
                   wf2_ref, bf2_ref, out_ref):
  dv = dinv_ref[...]
  acc = jnp.broadcast_to(b3_ref[...][None, :], (R_TC, 128)).astype(jnp.float32)
  for c in range(8):
    acc = acc + jnp.dot(s_ref[c] * dv[:, None], w3_ref[c * L:(c + 1) * L, :],
                        preferred_element_type=jnp.float32)
  h3 = jnp.maximum(acc, 0.0)
  h4 = jnp.maximum(
      jnp.dot(h3, wf1_ref[...], preferred_element_type=jnp.float32)
      + bf1_ref[...][None, :], 0.0)
  out_ref[...] = (jnp.dot(h4, wf2_ref[...], preferred_element_type=jnp.float32)
                  + bf2_ref[...][None, :])


def _tc_final(s3, dinv, w3, b3, wf1, bf1, wf2, bf2):
  return pl.pallas_call(
      _tc_final_body,
      grid=(G_TC,),
      in_specs=[
          pl.BlockSpec((8, R_TC, L), lambda i: (0, i, 0)),
          pl.BlockSpec((R_TC,), lambda i: (i,)),
          pl.BlockSpec((128, 128), lambda i: (0, 0)),
          pl.BlockSpec((128,), lambda i: (0,)),
          pl.BlockSpec((128, 64), lambda i: (0, 0)),
          pl.BlockSpec((64,), lambda i: (0,)),
          pl.BlockSpec((64, 2), lambda i: (0, 0)),
          pl.BlockSpec((2,), lambda i: (0,)),
      ],
      out_specs=pl.BlockSpec((R_TC, 2), lambda i: (i, 0)),
      out_shape=jax.ShapeDtypeStruct((N_PAD, 2), jnp.float32),
  )(s3, dinv, w3, b3, wf1, bf1, wf2, bf2)


_DEG_KERNEL = None
_PROP_KERNELS = {}


def _deg_kernel():
  global _DEG_KERNEL
  if _DEG_KERNEL is None:
    _DEG_KERNEL = _make_deg_kernel()
  return _DEG_KERNEL


def _prop_kernel(nc):
  if nc not in _PROP_KERNELS:
    _PROP_KERNELS[nc] = _make_prop_kernel(nc)
  return _PROP_KERNELS[nc]


@jax.jit
def kernel(x, edge_index, W1, b1, W2, b2, W3, b3, Wf1, bf1, Wf2, bf2):
  src = edge_index[0].astype(jnp.int32)
  dst = edge_index[1].astype(jnp.int32)
  n_extra = E_PAD - N_EDGES
  pad_src = jnp.zeros((n_extra,), jnp.int32)
  pad_dst = N_NODES + (jnp.arange(n_extra, dtype=jnp.int32) % 64)
  src3 = jnp.concatenate([src, pad_src]).reshape(NS, ROWS_PER_TILE, 128)
  dst3 = jnp.concatenate([dst, pad_dst]).reshape(NS, ROWS_PER_TILE, 128)
  sd = jnp.stack([src3, dst3], axis=2)

  x16 = jnp.zeros((N_PAD, L), jnp.float32).at[:N_NODES, :3].set(x)
  ones = jnp.ones((N_PAD,), jnp.float32)
  w1p = jnp.zeros((L, 64), jnp.float32).at[:3, :].set(W1)

  deg = _deg_kernel()(sd, ones)
  dinv, g1 = _tc_prep(x16, deg)
  s1 = _prop_kernel(1)(g1, sd)
  g2 = _tc_layer(s1, dinv, w1p, b1, g1=g1)
  s2 = _prop_kernel(4)(g2, sd)
  g3 = _tc_layer(s2, dinv, W2, b2)
  s3 = _prop_kernel(8)(g3, sd)
  out = _tc_final(s3, dinv, W3, b3, Wf1, bf1, Wf2, bf2)
  return out[:N_NODES]


# speedup vs baseline: 15.1396x; 1.1119x over previous
import functools

import jax
import jax.numpy as jnp
from jax import lax
from jax.experimental import pallas as pl
from jax.experimental.pallas import tpu as pltpu
from jax.experimental.pallas import tpu_sc as plsc

N_NODES = 100000
N_EDGES = 1600000

NC, NS, L = 2, 16, 16
N_PAD = 100352
E_PAD = 1605632
ROWS_PER_TILE = E_PAD // (NS * 128)
K_WIN = 4
N_WIN = ROWS_PER_TILE // K_WIN
NODES_PER_TILE = N_PAD // NS
R_TC = 2048
G_TC = N_PAD // R_TC


def _sc_mesh():
  return plsc.VectorSubcoreMesh(
      core_axis_name="c", subcore_axis_name="s",
      num_cores=NC, num_subcores=NS)


def _make_deg_kernel():
  @functools.partial(
      pl.kernel, mesh=_sc_mesh(),
      out_type=jax.ShapeDtypeStruct((NC, N_PAD), jnp.float32),
      compiler_params=pltpu.CompilerParams(use_tc_tiling_on_sc=False),
      scratch_types=[
          pltpu.VMEM((K_WIN, 2, 128), jnp.int32),
          pltpu.VMEM((K_WIN, 2, 128), jnp.int32),
          pltpu.VMEM((K_WIN, 128), jnp.float32),
          pltpu.VMEM_SHARED((N_PAD,), jnp.float32),
      ],
  )
  def deg_kernel(sd_hbm, ones_hbm, deg_hbm, didxa, didxb, onesbuf, acc):
    cid = lax.axis_index("c")
    sid = lax.axis_index("s")
    sl = pl.ds(sid * NODES_PER_TILE, NODES_PER_TILE)
    half = N_WIN // 2

    def scat(w, didx):
      pltpu.sync_copy(sd_hbm.at[sid, pl.ds(w * K_WIN, K_WIN)], didx)
      for r in range(K_WIN):
        pltpu.sync_copy(onesbuf.at[r], acc.at[didx.at[r, 1]], add=True)

    def run(w_base, out_j):
      for k in range(K_WIN):
        for i in range(128 // L):
          onesbuf[k, pl.ds(L * i, L)] = jnp.ones((L,), jnp.float32)
      pltpu.sync_copy(ones_hbm.at[sl], acc.at[sl])
      plsc.subcore_barrier()

      def pair(p, _):
        scat(w_base + 2 * p, didxa)
        scat(w_base + 2 * p + 1, didxb)
        return ()

      lax.fori_loop(0, half // 2, pair, ())
      if half % 2:
        scat(w_base + half - 1, didxa)
      plsc.subcore_barrier()
      pltpu.sync_copy(acc.at[sl], deg_hbm.at[out_j, sl])

    @pl.when(cid == 0)
    def _():
      run(0, 0)

    @pl.when(cid == 1)
    def _():
      run(half, 1)

  return deg_kernel


def _make_prop_kernel(nc):
  n_out = NC if nc == 1 else nc

  @functools.partial(
      pl.kernel, mesh=_sc_mesh(),
      out_type=jax.ShapeDtypeStruct((n_out, N_PAD, L), jnp.float32),
      compiler_params=pltpu.CompilerParams(use_tc_tiling_on_sc=False),
      scratch_types=[
          pltpu.VMEM((K_WIN, 2, 128), jnp.int32),
          pltpu.VMEM((K_WIN, 2, 128), jnp.int32),
          pltpu.VMEM((K_WIN, 2, 128), jnp.int32),
          pltpu.VMEM((K_WIN, 128, L), jnp.float32),
          pltpu.VMEM((K_WIN, 128, L), jnp.float32),
          pltpu.VMEM((K_WIN, 128, L), jnp.float32),
          pltpu.VMEM_SHARED((N_PAD, L), jnp.float32),
          pltpu.SemaphoreType.DMA,
          pltpu.SemaphoreType.DMA,
          pltpu.SemaphoreType.DMA,
      ],
  )
  def prop_kernel(g_hbm, sd_hbm, s_hbm,
                  idxa, idxb, idxc, rowsa, rowsb, rowsc, acc,
                  gsema, gsemb, gsemc):
    cid = lax.axis_index("c")
    sid = lax.axis_index("s")
    sl = pl.ds(sid * NODES_PER_TILE, NODES_PER_TILE)

    def fire(jc, w, idx, rows, gsem):
      pltpu.sync_copy(sd_hbm.at[sid, pl.ds(w * K_WIN, K_WIN)], idx)
      for r in range(K_WIN):
        pltpu.async_copy(g_hbm.at[jc].at[idx.at[r, 0]], rows.at[r], gsem)

    def drain_scatter(jc, idx, rows, gsem):
      for r in range(K_WIN):
        pltpu.make_async_copy(
            g_hbm.at[jc].at[idx.at[r, 0]], rows.at[r], gsem).wait()
      for r in range(K_WIN):
        pltpu.sync_copy(rows.at[r], acc.at[idx.at[r, 1]], add=True)

    def run(jc, w_base, n_win, out_j):
      pltpu.sync_copy(g_hbm.at[jc, sl], acc.at[sl])
      plsc.subcore_barrier()

      fire(jc, w_base, idxa, rowsa, gsema)
      fire(jc, w_base + 1, idxb, rowsb, gsemb)
      n_tri = (n_win - 2) // 3
      rem = (n_win - 2) % 3

      def triple(t, _):
        w = w_base + 3 * t
        fire(jc, w + 2, idxc, rowsc, gsemc)
        drain_scatter(jc, idxa, rowsa, gsema)
        fire(jc, w + 3, idxa, rowsa, gsema)
        drain_scatter(jc, idxb, rowsb, gsemb)
        fire(jc, w + 4, idxb, rowsb, gsemb)
        drain_scatter(jc, idxc, rowsc, gsemc)
        return ()

      lax.fori_loop(0, n_tri, triple, ())
      drain_scatter(jc, idxa, rowsa, gsema)
      drain_scatter(jc, idxb, rowsb, gsemb)
      for q in range(rem):
        w = w_base + 3 * n_tri + 2 + q
        fire(jc, w, idxa, rowsa, gsema)
        drain_scatter(jc, idxa, rowsa, gsema)
      plsc.subcore_barrier()
      pltpu.sync_copy(acc.at[sl], s_hbm.at[out_j, sl])

    if nc == 1:
      half = N_WIN // 2

      @pl.when(cid == 0)
      def _():
        run(0, 0, half, 0)

      @pl.when(cid == 1)
      def _():
        run(0, half, N_WIN - half, 1)
    else:
      for j in range(nc):
        @pl.when(cid == j % NC)
        def _(j=j):
          run(j, 0, N_WIN, j)

  return prop_kernel


def _tc_prep_body(x_ref, deg_ref, dinv_ref, g_ref):
  dv = lax.rsqrt(deg_ref[0] + deg_ref[1] - 1.0)
  dinv_ref[...] = dv
  g_ref[0] = x_ref[...] * dv[:, None]


def _tc_prep(x16, deg):
  return pl.pallas_call(
      _tc_prep_body,
      grid=(G_TC,),
      in_specs=[
          pl.BlockSpec((R_TC, L), lambda i: (i, 0)),
          pl.BlockSpec((NC, R_TC), lambda i: (0, i)),
      ],
      out_specs=[
          pl.BlockSpec((R_TC,), lambda i: (i,)),
          pl.BlockSpec((1, R_TC, L), lambda i: (0, i, 0)),
      ],
      out_shape=[
          jax.ShapeDtypeStruct((N_PAD,), jnp.float32),
          jax.ShapeDtypeStruct((1, N_PAD, L), jnp.float32),
      ],
  )(x16, deg)


def _tc_layer_body(ncin, ncout, partial, s_ref, dinv_ref, w_ref, b_ref,
                   g_ref, out_ref):
  dv = dinv_ref[...]
  acc = jnp.broadcast_to(b_ref[...][None, :], (R_TC, ncout * L)).astype(
      jnp.float32)
  for c in range(ncin):
    if partial:
      sc = s_ref[0] + s_ref[1] - g_ref[0]
    else:
      sc = s_ref[c]
    acc = acc + jnp.dot(sc * dv[:, None], w_ref[c * L:(c + 1) * L, :],
                        preferred_element_type=jnp.float32)
  g = jnp.maximum(acc, 0.0) * dv[:, None]
  for c in range(ncout):
    out_ref[c] = g[:, c * L:(c + 1) * L]


def _tc_layer(s, dinv, w, b, g1=None):
  partial = g1 is not None
  ncin = 1 if partial else s.shape[0]
  ncout = w.shape[1] // L
  nphys = s.shape[0]
  in_specs = [
      pl.BlockSpec((nphys, R_TC, L), lambda i: (0, i, 0)),
      pl.BlockSpec((R_TC,), lambda i: (i,)),
      pl.BlockSpec(w.shape, lambda i: (0, 0)),
      pl.BlockSpec(b.shape, lambda i: (0,)),
  ]
  args = [s, dinv, w, b]
  if partial:
    in_specs.append(pl.BlockSpec((1, R_TC, L), lambda i: (0, i, 0)))
    args.append(g1)
  else:
    in_specs.append(pl.BlockSpec((1, 8), lambda i: (0, 0)))
    args.append(jnp.zeros((1, 8), jnp.float32))
  return pl.pallas_call(
      functools.partial(_tc_layer_body, ncin, ncout, partial),
      grid=(G_TC,),
      in_specs=in_specs,
      out_specs=pl.BlockSpec((ncout, R_TC, L), lambda i: (0, i, 0)),
      out_shape=jax.ShapeDtypeStruct((ncout, N_PAD, L), jnp.float32),
  )(*args)


def _tc_final_body(s_ref, dinv_ref, w3_ref, b3_ref, wf1_ref, bf1_ref,
                   wf2_ref, bf2_ref, out_ref):
  dv = dinv_ref[...]
  acc = jnp.broadcast_to(b3_ref[...][None, :], (R_TC, 128)).astype(jnp.float32)
  for c in range(8):
    acc = acc + jnp.dot(s_ref[c] * dv[:, None], w3_ref[c * L:(c + 1) * L, :],
                        preferred_element_type=jnp.float32)
  h3 = jnp.maximum(acc, 0.0)
  h4 = jnp.maximum(
      jnp.dot(h3, wf1_ref[...], preferred_element_type=jnp.float32)
      + bf1_ref[...][None, :], 0.0)
  out_ref[...] = (jnp.dot(h4, wf2_ref[...], preferred_element_type=jnp.float32)
                  + bf2_ref[...][None, :])


def _tc_final(s3, dinv, w3, b3, wf1, bf1, wf2, bf2):
  return pl.pallas_call(
      _tc_final_body,
      grid=(G_TC,),
      in_specs=[
          pl.BlockSpec((8, R_TC, L), lambda i: (0, i, 0)),
          pl.BlockSpec((R_TC,), lambda i: (i,)),
          pl.BlockSpec((128, 128), lambda i: (0, 0)),
          pl.BlockSpec((128,), lambda i: (0,)),
          pl.BlockSpec((128, 64), lambda i: (0, 0)),
          pl.BlockSpec((64,), lambda i: (0,)),
          pl.BlockSpec((64, 2), lambda i: (0, 0)),
          pl.BlockSpec((2,), lambda i: (0,)),
      ],
      out_specs=pl.BlockSpec((R_TC, 2), lambda i: (i, 0)),
      out_shape=jax.ShapeDtypeStruct((N_PAD, 2), jnp.float32),
  )(s3, dinv, w3, b3, wf1, bf1, wf2, bf2)


_DEG_KERNEL = None
_PROP_KERNELS = {}


def _deg_kernel():
  global _DEG_KERNEL
  if _DEG_KERNEL is None:
    _DEG_KERNEL = _make_deg_kernel()
  return _DEG_KERNEL


def _prop_kernel(nc):
  if nc not in _PROP_KERNELS:
    _PROP_KERNELS[nc] = _make_prop_kernel(nc)
  return _PROP_KERNELS[nc]


@jax.jit
def kernel(x, edge_index, W1, b1, W2, b2, W3, b3, Wf1, bf1, Wf2, bf2):
  src = edge_index[0].astype(jnp.int32)
  dst = edge_index[1].astype(jnp.int32)
  n_extra = E_PAD - N_EDGES
  pad_src = jnp.zeros((n_extra,), jnp.int32)
  pad_dst = N_NODES + (jnp.arange(n_extra, dtype=jnp.int32) % 64)
  src3 = jnp.concatenate([src, pad_src]).reshape(NS, ROWS_PER_TILE, 128)
  dst3 = jnp.concatenate([dst, pad_dst]).reshape(NS, ROWS_PER_TILE, 128)
  sd = jnp.stack([src3, dst3], axis=2)

  x16 = jnp.zeros((N_PAD, L), jnp.float32).at[:N_NODES, :3].set(x)
  ones = jnp.ones((N_PAD,), jnp.float32)
  w1p = jnp.zeros((L, 64), jnp.float32).at[:3, :].set(W1)

  deg = _deg_kernel()(sd, ones)
  dinv, g1 = _tc_prep(x16, deg)
  s1 = _prop_kernel(1)(g1, sd)
  g2 = _tc_layer(s1, dinv, w1p, b1, g1=g1)
  s2 = _prop_kernel(4)(g2, sd)
  g3 = _tc_layer(s2, dinv, W2, b2)
  s3 = _prop_kernel(8)(g3, sd)
  out = _tc_final(s3, dinv, W3, b3, Wf1, bf1, Wf2, bf2)
  return out[:N_NODES]
